# baseline (speedup 1.0000x reference)
import functools
import jax
import jax.numpy as jnp
from jax import lax
from jax.experimental import pallas as pl
from jax.experimental.pallas import tpu as pltpu
from jax.experimental.pallas import tpu_sc as plsc

R, C = 128, 100000
NC, NS = 2, 16
NW = NC * NS
ROWS_PER_W = R // NW
CHUNK = 20000
NCHUNK = C // CHUNK
NVEC = CHUNK // 16
U = 10
NITER = NVEC // U

NEG_INF = float("-inf")
BIG = 1 << 30


def _merge(va, ia, vb, ib):
    take_b = (vb > va) | ((vb == va) & (ib < ia))
    return jnp.where(take_b, vb, va), jnp.where(take_b, ib, ia)


def _lane_perm(x, idx):
    return lax.gather(
        x, idx[:, None],
        dimension_numbers=lax.GatherDimensionNumbers(
            offset_dims=(), collapsed_slice_dims=(0,), start_index_map=(0,)),
        slice_sizes=(1,),
        mode=lax.GatherScatterMode.PROMISE_IN_BOUNDS)


def _sc_body(x_hbm, out_hbm, buf0, buf1, res_v, sem0, sem1):
    cid = lax.axis_index("c")
    sid = lax.axis_index("s")
    wid = sid * NC + cid
    base_row = wid * ROWS_PER_W
    lanes = lax.iota(jnp.int32, 16)

    bufs = (buf0, buf1)
    sems = (sem0, sem1)
    results = jnp.zeros((16,), jnp.int32)

    for r in range(ROWS_PER_W):
        row = base_row + r
        rbase = row * C
        pending = [None, None]
        pending[0] = pltpu.make_async_copy(
            x_hbm.at[pl.ds(rbase, CHUNK)], bufs[0], sems[0])
        pending[0].start()

        vals = [jnp.full((16,), NEG_INF, jnp.float32) for _ in range(U)]
        vecs = [jnp.zeros((16,), jnp.int32) for _ in range(U)]

        for ch in range(NCHUNK):
            buf = bufs[ch % 2]
            pending[ch % 2].wait()
            if ch + 1 < NCHUNK:
                pending[(ch + 1) % 2] = pltpu.make_async_copy(
                    x_hbm.at[pl.ds(rbase + (ch + 1) * CHUNK, CHUNK)],
                    bufs[(ch + 1) % 2], sems[(ch + 1) % 2])
                pending[(ch + 1) % 2].start()

            def step(i, carry):
                cvals, cvecs = carry
                gidx = ch * NVEC + i * U
                new_vals, new_vecs = [], []
                for j in range(U):
                    v = buf[pl.ds((i * U + j) * 16, 16)]
                    m = v > cvals[j]
                    new_vals.append(jnp.where(m, v, cvals[j]))
                    new_vecs.append(jnp.where(m, gidx + j, cvecs[j]))
                return tuple(new_vals), tuple(new_vecs)

            vals, vecs = lax.fori_loop(
                0, NITER, step, (tuple(vals), tuple(vecs)))
            vals, vecs = list(vals), list(vecs)

        bv = vals[0]
        bi = vecs[0] * 16 + lanes
        for j in range(1, U):
            bv, bi = _merge(bv, bi, vals[j], vecs[j] * 16 + lanes)

        for s in (8, 4, 2, 1):
            perm = lanes ^ s
            bv2 = _lane_perm(bv, perm)
            bi2 = _lane_perm(bi, perm)
            bv, bi = _merge(bv, bi, bv2, bi2)
        results = jnp.where(lanes == r, bi, results)

    res_v[...] = results
    pltpu.sync_copy(res_v, out_hbm.at[pl.ds(wid * 16, 16)])


def kernel(x):
    mesh = plsc.VectorSubcoreMesh(core_axis_name="c", subcore_axis_name="s")
    kern = pl.kernel(
        _sc_body,
        mesh=mesh,
        out_type=jax.ShapeDtypeStruct((NW * 16,), jnp.int32),
        scratch_types=[
            pltpu.VMEM((CHUNK,), jnp.float32),
            pltpu.VMEM((CHUNK,), jnp.float32),
            pltpu.VMEM((16,), jnp.int32),
            pltpu.SemaphoreType.DMA,
            pltpu.SemaphoreType.DMA,
        ],
    )
    out = kern(x.reshape(-1))
    return out.reshape(NW, 16)[:, :ROWS_PER_W].reshape(R).astype(jnp.int64)


# speedup vs baseline: 1.7704x; 1.7704x over previous
import jax
import jax.numpy as jnp
from jax import lax
from jax.experimental import pallas as pl
from jax.experimental.pallas import tpu as pltpu
from jax.experimental.pallas import tpu_sc as plsc

R, C = 128, 100000
NC, NS = 2, 16
C_SPLIT = 59904
BLK_R = 8
N_BLK = R // BLK_R

CK_BIG = 6656
_OFFS_H0 = (59904, 66560, 73216, 79872)
_OFFS_H1 = (80000, 86656, 93312, 99840)
_SIZES = (CK_BIG, CK_BIG, CK_BIG, 128)
C_TAIL = 99968

NEG_INF = float("-inf")
BIG = 1 << 30


def _merge(va, ia, vb, ib):
    take_b = (vb > va) | ((vb == va) & (ib < ia))
    return jnp.where(take_b, vb, va), jnp.where(take_b, ib, ia)


def _lane_perm(x, idx):
    return lax.gather(
        x, idx[:, None],
        dimension_numbers=lax.GatherDimensionNumbers(
            offset_dims=(), collapsed_slice_dims=(0,), start_index_map=(0,)),
        slice_sizes=(1,),
        mode=lax.GatherScatterMode.PROMISE_IN_BOUNDS)


def _scan_rows(buf, ck, c0, lanes, run):
    nv = ck // 16
    u = 8 if nv % 8 == 0 else (2 if nv % 2 == 0 else 1)
    iters = nv // u
    for r8 in range(BLK_R):
        vals = [jnp.full((16,), NEG_INF, jnp.float32) for _ in range(u)]
        vecs = [jnp.zeros((16,), jnp.int32) for _ in range(u)]

        def step(i, carry, r8=r8):
            cvals, cvecs = carry
            base = i * u
            nvals, nvecs = [], []
            for j in range(u):
                v = buf[r8, pl.ds((base + j) * 16, 16)]
                m = v > cvals[j]
                nvals.append(jnp.where(m, v, cvals[j]))
                nvecs.append(jnp.where(m, base + j, cvecs[j]))
            return tuple(nvals), tuple(nvecs)

        if iters == 1:
            vals, vecs = step(0, (tuple(vals), tuple(vecs)))
        else:
            vals, vecs = lax.fori_loop(0, iters, step,
                                       (tuple(vals), tuple(vecs)))
        bv = vals[0]
        bi = vecs[0] * 16 + lanes + c0
        for j in range(1, u):
            bv, bi = _merge(bv, bi, vals[j], vecs[j] * 16 + lanes + c0)
        run[r8] = _merge(run[r8][0], run[r8][1], bv, bi)
    return run


def _sc_body(x_hbm, oval_hbm, oidx_hbm, bufa, bufb, bufc,
             res_v, res_i, sema, semb, semc):
    cid = lax.axis_index("c")
    sid = lax.axis_index("s")
    wid = sid * NC + cid
    b = wid // 2
    h = wid % 2
    row0 = b * BLK_R
    lanes = lax.iota(jnp.int32, 16)

    def off(k):
        o = jnp.where(h == 0, _OFFS_H0[k], _OFFS_H1[k])
        return pl.multiple_of(o, 128)

    offs = [off(k) for k in range(4)]
    cpa0 = pltpu.make_async_copy(
        x_hbm.at[pl.ds(row0, 8), pl.ds(offs[0], CK_BIG)], bufa, sema)
    cpa0.start()
    cpb = pltpu.make_async_copy(
        x_hbm.at[pl.ds(row0, 8), pl.ds(offs[1], CK_BIG)], bufb, semb)
    cpb.start()
    cpc = pltpu.make_async_copy(
        x_hbm.at[pl.ds(row0, 8), pl.ds(offs[3], 128)], bufc, semc)
    cpc.start()

    run = [(jnp.full((16,), NEG_INF, jnp.float32),
            jnp.full((16,), BIG, jnp.int32)) for _ in range(BLK_R)]

    cpa0.wait()
    run = _scan_rows(bufa, CK_BIG, offs[0], lanes, run)
    cpa1 = pltpu.make_async_copy(
        x_hbm.at[pl.ds(row0, 8), pl.ds(offs[2], CK_BIG)], bufa, sema)
    cpa1.start()
    cpb.wait()
    run = _scan_rows(bufb, CK_BIG, offs[1], lanes, run)
    cpa1.wait()
    run = _scan_rows(bufa, CK_BIG, offs[2], lanes, run)
    cpc.wait()
    run = _scan_rows(bufc, 128, offs[3], lanes, run)

    val_res = jnp.zeros((16,), jnp.float32)
    idx_res = jnp.zeros((16,), jnp.int32)
    for r8 in range(BLK_R):
        bv, bi = run[r8]
        for s in (8, 4, 2, 1):
            perm = lanes ^ s
            bv, bi = _merge(bv, bi, _lane_perm(bv, perm), _lane_perm(bi, perm))
        val_res = jnp.where(lanes == r8, bv, val_res)
        idx_res = jnp.where(lanes == r8, bi, idx_res)

    res_v[...] = val_res
    res_i[...] = idx_res
    obase = pl.multiple_of(h * (N_BLK * 16) + b * 16, 16)
    pltpu.sync_copy(res_v, oval_hbm.at[pl.ds(obase, 16)])
    pltpu.sync_copy(res_i, oidx_hbm.at[pl.ds(obase, 16)])


def _tc_body(x_ref, tail_ref, oval_ref, oidx_ref):
    x = x_ref[...]
    m = jnp.max(x, axis=1, keepdims=True)
    idx = lax.broadcasted_iota(jnp.int32, (BLK_R, C_SPLIT), 1)
    cand = jnp.where(x == m, idx, jnp.int32(BIG))
    mi = jnp.min(cand, axis=1)
    mv = m[:, 0]

    tcol = lax.broadcasted_iota(jnp.int32, (BLK_R, 128), 1)
    t = jnp.where(tcol < (C - C_TAIL), tail_ref[...], jnp.float32(NEG_INF))
    tm = jnp.max(t, axis=1, keepdims=True)
    tcand = jnp.where(t == tm, tcol + C_TAIL, jnp.int32(BIG))
    ti = jnp.min(tcand, axis=1)
    tv = tm[:, 0]

    take_t = tv > mv
    oidx_ref[0, 0, :] = jnp.where(take_t, ti, mi)
    oval_ref[0, 0, :] = jnp.where(take_t, tv, mv)


def kernel(x):
    mesh = plsc.VectorSubcoreMesh(core_axis_name="c", subcore_axis_name="s")
    sc_kern = pl.kernel(
        _sc_body,
        mesh=mesh,
        out_type=(
            jax.ShapeDtypeStruct((2 * N_BLK * 16,), jnp.float32),
            jax.ShapeDtypeStruct((2 * N_BLK * 16,), jnp.int32),
        ),
        scratch_types=[
            pltpu.VMEM((BLK_R, CK_BIG), jnp.float32),
            pltpu.VMEM((BLK_R, CK_BIG), jnp.float32),
            pltpu.VMEM((BLK_R, 128), jnp.float32),
            pltpu.VMEM((16,), jnp.float32),
            pltpu.VMEM((16,), jnp.int32),
            pltpu.SemaphoreType.DMA,
            pltpu.SemaphoreType.DMA,
            pltpu.SemaphoreType.DMA,
        ],
    )
    sval, sidx = sc_kern(x)

    tval, tidx = pl.pallas_call(
        _tc_body,
        grid=(N_BLK,),
        in_specs=[
            pl.BlockSpec((BLK_R, C_SPLIT), lambda i: (i, 0)),
            pl.BlockSpec((BLK_R, 128), lambda i: (i, C_TAIL // 128)),
        ],
        out_specs=[
            pl.BlockSpec((1, 1, BLK_R), lambda i: (i, 0, 0)),
            pl.BlockSpec((1, 1, BLK_R), lambda i: (i, 0, 0)),
        ],
        out_shape=[
            jax.ShapeDtypeStruct((N_BLK, 1, BLK_R), jnp.float32),
            jax.ShapeDtypeStruct((N_BLK, 1, BLK_R), jnp.int32),
        ],
    )(x, x)

    tval = tval.reshape(R)
    tidx = tidx.reshape(R)
    sv = sval.reshape(2, N_BLK, 16)[:, :, :BLK_R].reshape(2, R)
    si = sidx.reshape(2, N_BLK, 16)[:, :, :BLK_R].reshape(2, R)

    v, i = tval, tidx
    for k in (0, 1):
        t = (sv[k] > v) | ((sv[k] == v) & (si[k] < i))
        v = jnp.where(t, sv[k], v)
        i = jnp.where(t, si[k], i)
    return i.astype(jnp.int64)


# speedup vs baseline: 1.7891x; 1.0105x over previous
import jax
import jax.numpy as jnp
from jax import lax
from jax.experimental import pallas as pl
from jax.experimental.pallas import tpu as pltpu
from jax.experimental.pallas import tpu_sc as plsc

R, C = 128, 100000
NC, NS = 2, 16
BLK_R = 8
N_BLK = R // BLK_R

C_SPLIT = 50048
TC_W = C_SPLIT
TC_VALID = C - C_SPLIT

CK = 6272
_OFFS_H0 = (0, 6272, 12544, 18816)
_OFFS_H1 = (25088, 31360, 37632, 43776)

NEG_INF = float("-inf")
BIG = 1 << 30


def _merge(va, ia, vb, ib):
    take_b = (vb > va) | ((vb == va) & (ib < ia))
    return jnp.where(take_b, vb, va), jnp.where(take_b, ib, ia)


def _lane_perm(x, idx):
    return lax.gather(
        x, idx[:, None],
        dimension_numbers=lax.GatherDimensionNumbers(
            offset_dims=(), collapsed_slice_dims=(0,), start_index_map=(0,)),
        slice_sizes=(1,),
        mode=lax.GatherScatterMode.PROMISE_IN_BOUNDS)


def _scan_rows(buf, ck, c0, lanes, run):
    nv = ck // 16
    u = 8 if nv % 8 == 0 else (2 if nv % 2 == 0 else 1)
    iters = nv // u
    for r8 in range(BLK_R):
        vals = [jnp.full((16,), NEG_INF, jnp.float32) for _ in range(u)]
        vecs = [jnp.zeros((16,), jnp.int32) for _ in range(u)]

        def step(i, carry, r8=r8):
            cvals, cvecs = carry
            base = i * u
            nvals, nvecs = [], []
            for j in range(u):
                v = buf[r8, pl.ds((base + j) * 16, 16)]
                m = v > cvals[j]
                nvals.append(jnp.where(m, v, cvals[j]))
                nvecs.append(jnp.where(m, base + j, cvecs[j]))
            return tuple(nvals), tuple(nvecs)

        if iters == 1:
            vals, vecs = step(0, (tuple(vals), tuple(vecs)))
        else:
            vals, vecs = lax.fori_loop(0, iters, step,
                                       (tuple(vals), tuple(vecs)))
        bv = vals[0]
        bi = vecs[0] * 16 + lanes + c0
        for j in range(1, u):
            bv, bi = _merge(bv, bi, vals[j], vecs[j] * 16 + lanes + c0)
        run[r8] = _merge(run[r8][0], run[r8][1], bv, bi)
    return run


def _sc_body(x_hbm, oval_hbm, oidx_hbm, bufa, bufb,
             res_v, res_i, sema, semb):
    cid = lax.axis_index("c")
    sid = lax.axis_index("s")
    wid = sid * NC + cid
    b = wid // 2
    h = wid % 2
    row0 = b * BLK_R
    lanes = lax.iota(jnp.int32, 16)

    def off(k):
        o = jnp.where(h == 0, _OFFS_H0[k], _OFFS_H1[k])
        return pl.multiple_of(o, 128)

    offs = [off(k) for k in range(4)]
    bufs = (bufa, bufb)
    sems = (sema, semb)
    pend = [None, None]
    pend[0] = pltpu.make_async_copy(
        x_hbm.at[pl.ds(row0, 8), pl.ds(offs[0], CK)], bufs[0], sems[0])
    pend[0].start()
    pend[1] = pltpu.make_async_copy(
        x_hbm.at[pl.ds(row0, 8), pl.ds(offs[1], CK)], bufs[1], sems[1])
    pend[1].start()

    run = [(jnp.full((16,), NEG_INF, jnp.float32),
            jnp.full((16,), BIG, jnp.int32)) for _ in range(BLK_R)]

    for k in range(4):
        pend[k % 2].wait()
        run = _scan_rows(bufs[k % 2], CK, offs[k], lanes, run)
        if k + 2 < 4:
            pend[k % 2] = pltpu.make_async_copy(
                x_hbm.at[pl.ds(row0, 8), pl.ds(offs[k + 2], CK)],
                bufs[k % 2], sems[k % 2])
            pend[k % 2].start()

    val_res = jnp.zeros((16,), jnp.float32)
    idx_res = jnp.zeros((16,), jnp.int32)
    for r8 in range(BLK_R):
        bv, bi = run[r8]
        for s in (8, 4, 2, 1):
            perm = lanes ^ s
            bv, bi = _merge(bv, bi, _lane_perm(bv, perm), _lane_perm(bi, perm))
        val_res = jnp.where(lanes == r8, bv, val_res)
        idx_res = jnp.where(lanes == r8, bi, idx_res)

    res_v[...] = val_res
    res_i[...] = idx_res
    obase = pl.multiple_of(h * (N_BLK * 16) + b * 16, 16)
    pltpu.sync_copy(res_v, oval_hbm.at[pl.ds(obase, 16)])
    pltpu.sync_copy(res_i, oidx_hbm.at[pl.ds(obase, 16)])


def _tc_body(x_ref, oval_ref, oidx_ref):
    col = lax.broadcasted_iota(jnp.int32, (BLK_R, TC_W), 1)
    x = jnp.where(col < TC_VALID, x_ref[...], jnp.float32(NEG_INF))
    m = jnp.max(x, axis=1, keepdims=True)
    cand = jnp.where(x == m, col + C_SPLIT, jnp.int32(BIG))
    oidx_ref[0, 0, :] = jnp.min(cand, axis=1)
    oval_ref[0, 0, :] = m[:, 0]


def kernel(x):
    mesh = plsc.VectorSubcoreMesh(core_axis_name="c", subcore_axis_name="s")
    sc_kern = pl.kernel(
        _sc_body,
        mesh=mesh,
        compiler_params=pltpu.CompilerParams(use_tc_tiling_on_sc=True),
        out_type=(
            jax.ShapeDtypeStruct((2 * N_BLK * 16,), jnp.float32),
            jax.ShapeDtypeStruct((2 * N_BLK * 16,), jnp.int32),
        ),
        scratch_types=[
            pltpu.VMEM((BLK_R, CK), jnp.float32),
            pltpu.VMEM((BLK_R, CK), jnp.float32),
            pltpu.VMEM((16,), jnp.float32),
            pltpu.VMEM((16,), jnp.int32),
            pltpu.SemaphoreType.DMA,
            pltpu.SemaphoreType.DMA,
        ],
    )
    sval, sidx = sc_kern(x)

    tval, tidx = pl.pallas_call(
        _tc_body,
        grid=(N_BLK,),
        in_specs=[pl.BlockSpec((BLK_R, TC_W), lambda i: (i, 1))],
        out_specs=[
            pl.BlockSpec((1, 1, BLK_R), lambda i: (i, 0, 0)),
            pl.BlockSpec((1, 1, BLK_R), lambda i: (i, 0, 0)),
        ],
        out_shape=[
            jax.ShapeDtypeStruct((N_BLK, 1, BLK_R), jnp.float32),
            jax.ShapeDtypeStruct((N_BLK, 1, BLK_R), jnp.int32),
        ],
    )(x)

    tval = tval.reshape(R)
    tidx = tidx.reshape(R)
    sv = sval.reshape(2, N_BLK, 16)[:, :, :BLK_R].reshape(2, R)
    si = sidx.reshape(2, N_BLK, 16)[:, :, :BLK_R].reshape(2, R)

    v, i = sv[0], si[0]
    for vb, ib in ((sv[1], si[1]), (tval, tidx)):
        t = (vb > v) | ((vb == v) & (ib < i))
        v = jnp.where(t, vb, v)
        i = jnp.where(t, ib, i)
    return i.astype(jnp.int64)


# speedup vs baseline: 4.1043x; 2.2941x over previous
import jax
import jax.numpy as jnp
from jax import lax
from jax.experimental import pallas as pl
from jax.experimental.pallas import tpu as pltpu
from jax.experimental.pallas import tpu_sc as plsc

R, C = 128, 100000
NC, NS = 2, 16
NW = NC * NS

T_SPLIT = 52000
Q = 1632
Q_LAST0 = T_SPLIT - Q
CKT = 408
NCH = Q // CKT

TC_BW = 4000
TC_BLK0 = T_SPLIT // TC_BW
TC_NBLK = (C - T_SPLIT) // TC_BW
A = 10
SUB = TC_BW // 8

NEG_INF = float("-inf")
BIG = 1 << 30


def _merge(va, ia, vb, ib):
    take_b = (vb > va) | ((vb == va) & (ib < ia))
    return jnp.where(take_b, vb, va), jnp.where(take_b, ib, ia)


def _sc_body(y_hbm, oval_hbm, oidx_hbm,
             bufa, bufb, stage_v, stage_i, mrg_v, mrg_i, shv, shi,
             sema, semb):
    cid = lax.axis_index("c")
    sid = lax.axis_index("s")
    wid = sid * NC + cid
    base = jnp.minimum(wid * Q, Q_LAST0)
    base = pl.multiple_of(base, 8)
    lanes = lax.iota(jnp.int32, 16)

    bufs = (bufa, bufb)
    sems = (sema, semb)
    pend = [None, None]
    pend[0] = pltpu.make_async_copy(
        y_hbm.at[pl.ds(base, CKT), :], bufs[0], sems[0])
    pend[0].start()
    pend[1] = pltpu.make_async_copy(
        y_hbm.at[pl.ds(base + CKT, CKT), :], bufs[1], sems[1])
    pend[1].start()

    gv = [jnp.full((16,), NEG_INF, jnp.float32) for _ in range(8)]
    gi = [jnp.full((16,), BIG, jnp.int32) for _ in range(8)]

    for k in range(NCH):
        pend[k % 2].wait()
        buf = bufs[k % 2]
        t0 = base + k * CKT

        def step(t, carry, buf=buf, t0=t0):
            cv, ci = carry
            nv, ni = [], []
            col = t0 + t
            for g in range(8):
                v = buf[t, pl.ds(g * 16, 16)]
                m = v > cv[g]
                nv.append(jnp.where(m, v, cv[g]))
                ni.append(jnp.where(m, col, ci[g]))
            return tuple(nv), tuple(ni)

        gv, gi = lax.fori_loop(0, CKT, step, (tuple(gv), tuple(gi)))
        gv, gi = list(gv), list(gi)
        if k + 2 < NCH:
            pend[k % 2] = pltpu.make_async_copy(
                y_hbm.at[pl.ds(base + (k + 2) * CKT, CKT), :],
                bufs[k % 2], sems[k % 2])
            pend[k % 2].start()

    for g in range(8):
        stage_v[pl.ds(g * 16, 16)] = gv[g]
        stage_i[pl.ds(g * 16, 16)] = gi[g]
    sbase = pl.multiple_of(sid * 16, 16)
    for g in range(8):
        pltpu.sync_copy(stage_v.at[pl.ds(g * 16, 16)],
                        shv.at[pl.ds(g * 256 + sbase, 16)])
        pltpu.sync_copy(stage_i.at[pl.ds(g * 16, 16)],
                        shi.at[pl.ds(g * 256 + sbase, 16)])
    plsc.subcore_barrier()

    @pl.when(sid < 8)
    def _():
        gbase = pl.multiple_of(sid * 256, 16)
        pltpu.sync_copy(shv.at[pl.ds(gbase, 256)], mrg_v)
        pltpu.sync_copy(shi.at[pl.ds(gbase, 256)], mrg_i)
        bv = mrg_v[pl.ds(0, 16)]
        bi = mrg_i[pl.ds(0, 16)]
        for s in range(1, 16):
            bv, bi = _merge(bv, bi, mrg_v[pl.ds(s * 16, 16)],
                            mrg_i[pl.ds(s * 16, 16)])
        stage_v[pl.ds(0, 16)] = bv
        stage_i[pl.ds(0, 16)] = bi
        obase = pl.multiple_of(cid * R + sid * 16, 16)
        pltpu.sync_copy(stage_v.at[pl.ds(0, 16)],
                        oval_hbm.at[pl.ds(obase, 16)])
        pltpu.sync_copy(stage_i.at[pl.ds(0, 16)],
                        oidx_hbm.at[pl.ds(obase, 16)])


def _tc_body(y_ref, oval_ref, oidx_ref):
    i = pl.program_id(0)
    tb = (i + TC_BLK0) * TC_BW

    accv = [None] * A
    accj = [None] * A
    for j in range(SUB):
        a = j % A
        v = y_ref[pl.ds(j * 8, 8), :]
        if accv[a] is None:
            accv[a] = v
            accj[a] = jnp.full((8, 128), j, jnp.int32)
        else:
            m = v > accv[a]
            accv[a] = jnp.where(m, v, accv[a])
            accj[a] = jnp.where(m, jnp.int32(j), accj[a])

    bv, bj = accv[0], accj[0]
    for a in range(1, A):
        t = (accv[a] > bv) | ((accv[a] == bv) & (accj[a] < bj))
        bv = jnp.where(t, accv[a], bv)
        bj = jnp.where(t, accj[a], bj)

    sub = lax.broadcasted_iota(jnp.int32, (8, 128), 0)
    bc = bj * 8 + sub + tb
    vmax = jnp.max(bv, axis=0)
    cand = jnp.where(bv == vmax[None, :], bc, jnp.int32(BIG))
    cmin = jnp.min(cand, axis=0)

    @pl.when(i == 0)
    def _():
        oval_ref[0, 0, :] = vmax
        oidx_ref[0, 0, :] = cmin

    @pl.when(i > 0)
    def _():
        pv = oval_ref[0, 0, :]
        pi = oidx_ref[0, 0, :]
        t = (vmax > pv) | ((vmax == pv) & (cmin < pi))
        oval_ref[0, 0, :] = jnp.where(t, vmax, pv)
        oidx_ref[0, 0, :] = jnp.where(t, cmin, pi)


def kernel(x):
    y = jnp.transpose(x)

    mesh = plsc.VectorSubcoreMesh(core_axis_name="c", subcore_axis_name="s")
    sc_kern = pl.kernel(
        _sc_body,
        mesh=mesh,
        compiler_params=pltpu.CompilerParams(use_tc_tiling_on_sc=True),
        out_type=(
            jax.ShapeDtypeStruct((NC * R,), jnp.float32),
            jax.ShapeDtypeStruct((NC * R,), jnp.int32),
        ),
        scratch_types=[
            pltpu.VMEM((CKT, 128), jnp.float32),
            pltpu.VMEM((CKT, 128), jnp.float32),
            pltpu.VMEM((128,), jnp.float32),
            pltpu.VMEM((128,), jnp.int32),
            pltpu.VMEM((256,), jnp.float32),
            pltpu.VMEM((256,), jnp.int32),
            pltpu.VMEM_SHARED((2048,), jnp.float32),
            pltpu.VMEM_SHARED((2048,), jnp.int32),
            pltpu.SemaphoreType.DMA,
            pltpu.SemaphoreType.DMA,
        ],
    )
    sval, sidx = sc_kern(y)

    tval, tidx = pl.pallas_call(
        _tc_body,
        grid=(TC_NBLK,),
        in_specs=[pl.BlockSpec((TC_BW, 128), lambda i: (i + TC_BLK0, 0))],
        out_specs=[
            pl.BlockSpec((1, 1, 128), lambda i: (0, 0, 0)),
            pl.BlockSpec((1, 1, 128), lambda i: (0, 0, 0)),
        ],
        out_shape=[
            jax.ShapeDtypeStruct((1, 1, 128), jnp.float32),
            jax.ShapeDtypeStruct((1, 1, 128), jnp.int32),
        ],
    )(y)

    tv = tval.reshape(R)
    ti = tidx.reshape(R)
    sv = sval.reshape(NC, R)
    si = sidx.reshape(NC, R)

    v, i = sv[0], si[0]
    for vb, ib in ((sv[1], si[1]), (tv, ti)):
        t = (vb > v) | ((vb == v) & (ib < i))
        v = jnp.where(t, vb, v)
        i = jnp.where(t, ib, i)
    return i.astype(jnp.int64)


# speedup vs baseline: 4.2935x; 1.0461x over previous
import jax
import jax.numpy as jnp
from jax import lax
from jax.experimental import pallas as pl
from jax.experimental.pallas import tpu as pltpu
from jax.experimental.pallas import tpu_sc as plsc

R, C = 128, 100000
NC, NS = 2, 16
NW = NC * NS

T_SPLIT = 28000
Q = 896
Q_LAST0 = T_SPLIT - Q
CKT = 224
NCH = Q // CKT

TC_BW = 4000
TC_BLK0 = T_SPLIT // TC_BW
TC_NBLK = (C - T_SPLIT) // TC_BW
A = 10
SUB = TC_BW // 8

NEG_INF = float("-inf")
BIG = 1 << 30


def _merge(va, ia, vb, ib):
    take_b = (vb > va) | ((vb == va) & (ib < ia))
    return jnp.where(take_b, vb, va), jnp.where(take_b, ib, ia)


def _sc_body(y_hbm, oval_hbm, oidx_hbm,
             bufa, bufb, stage_v, stage_i, mrg_v, mrg_i, shv, shi,
             sema, semb):
    cid = lax.axis_index("c")
    sid = lax.axis_index("s")
    wid = sid * NC + cid
    base = jnp.minimum(wid * Q, Q_LAST0)
    base = pl.multiple_of(base, 8)
    lanes = lax.iota(jnp.int32, 16)

    bufs = (bufa, bufb)
    sems = (sema, semb)
    pend = [None, None]
    pend[0] = pltpu.make_async_copy(
        y_hbm.at[pl.ds(base, CKT), :], bufs[0], sems[0])
    pend[0].start()
    pend[1] = pltpu.make_async_copy(
        y_hbm.at[pl.ds(base + CKT, CKT), :], bufs[1], sems[1])
    pend[1].start()

    gv = [jnp.full((16,), NEG_INF, jnp.float32) for _ in range(8)]
    gi = [jnp.full((16,), BIG, jnp.int32) for _ in range(8)]

    for k in range(NCH):
        pend[k % 2].wait()
        buf = bufs[k % 2]
        t0 = base + k * CKT

        def step(t, carry, buf=buf, t0=t0):
            cv, ci = carry
            nv, ni = [], []
            col = t0 + t
            for g in range(8):
                v = buf[t, pl.ds(g * 16, 16)]
                m = v > cv[g]
                nv.append(jnp.where(m, v, cv[g]))
                ni.append(jnp.where(m, col, ci[g]))
            return tuple(nv), tuple(ni)

        gv, gi = lax.fori_loop(0, CKT, step, (tuple(gv), tuple(gi)))
        gv, gi = list(gv), list(gi)
        if k + 2 < NCH:
            pend[k % 2] = pltpu.make_async_copy(
                y_hbm.at[pl.ds(base + (k + 2) * CKT, CKT), :],
                bufs[k % 2], sems[k % 2])
            pend[k % 2].start()

    for g in range(8):
        stage_v[pl.ds(g * 16, 16)] = gv[g]
        stage_i[pl.ds(g * 16, 16)] = gi[g]
    sbase = pl.multiple_of(sid * 16, 16)
    for g in range(8):
        pltpu.sync_copy(stage_v.at[pl.ds(g * 16, 16)],
                        shv.at[pl.ds(g * 256 + sbase, 16)])
        pltpu.sync_copy(stage_i.at[pl.ds(g * 16, 16)],
                        shi.at[pl.ds(g * 256 + sbase, 16)])
    plsc.subcore_barrier()

    @pl.when(sid < 8)
    def _():
        gbase = pl.multiple_of(sid * 256, 16)
        pltpu.sync_copy(shv.at[pl.ds(gbase, 256)], mrg_v)
        pltpu.sync_copy(shi.at[pl.ds(gbase, 256)], mrg_i)
        bv = mrg_v[pl.ds(0, 16)]
        bi = mrg_i[pl.ds(0, 16)]
        for s in range(1, 16):
            bv, bi = _merge(bv, bi, mrg_v[pl.ds(s * 16, 16)],
                            mrg_i[pl.ds(s * 16, 16)])
        stage_v[pl.ds(0, 16)] = bv
        stage_i[pl.ds(0, 16)] = bi
        obase = pl.multiple_of(cid * R + sid * 16, 16)
        pltpu.sync_copy(stage_v.at[pl.ds(0, 16)],
                        oval_hbm.at[pl.ds(obase, 16)])
        pltpu.sync_copy(stage_i.at[pl.ds(0, 16)],
                        oidx_hbm.at[pl.ds(obase, 16)])


def _tc_body(y_ref, oval_ref, oidx_ref):
    i = pl.program_id(0)
    tb = (i + TC_BLK0) * TC_BW

    accv = [None] * A
    accj = [None] * A
    for j in range(SUB):
        a = j % A
        v = y_ref[pl.ds(j * 8, 8), :]
        if accv[a] is None:
            accv[a] = v
            accj[a] = jnp.full((8, 128), j, jnp.int32)
        else:
            m = v > accv[a]
            accv[a] = jnp.where(m, v, accv[a])
            accj[a] = jnp.where(m, jnp.int32(j), accj[a])

    bv, bj = accv[0], accj[0]
    for a in range(1, A):
        t = (accv[a] > bv) | ((accv[a] == bv) & (accj[a] < bj))
        bv = jnp.where(t, accv[a], bv)
        bj = jnp.where(t, accj[a], bj)

    sub = lax.broadcasted_iota(jnp.int32, (8, 128), 0)
    bc = bj * 8 + sub + tb
    vmax = jnp.max(bv, axis=0)
    cand = jnp.where(bv == vmax[None, :], bc, jnp.int32(BIG))
    cmin = jnp.min(cand, axis=0)

    @pl.when(i == 0)
    def _():
        oval_ref[0, 0, :] = vmax
        oidx_ref[0, 0, :] = cmin

    @pl.when(i > 0)
    def _():
        pv = oval_ref[0, 0, :]
        pi = oidx_ref[0, 0, :]
        t = (vmax > pv) | ((vmax == pv) & (cmin < pi))
        oval_ref[0, 0, :] = jnp.where(t, vmax, pv)
        oidx_ref[0, 0, :] = jnp.where(t, cmin, pi)


def kernel(x):
    y = jnp.transpose(x)

    mesh = plsc.VectorSubcoreMesh(core_axis_name="c", subcore_axis_name="s")
    sc_kern = pl.kernel(
        _sc_body,
        mesh=mesh,
        compiler_params=pltpu.CompilerParams(use_tc_tiling_on_sc=True),
        out_type=(
            jax.ShapeDtypeStruct((NC * R,), jnp.float32),
            jax.ShapeDtypeStruct((NC * R,), jnp.int32),
        ),
        scratch_types=[
            pltpu.VMEM((CKT, 128), jnp.float32),
            pltpu.VMEM((CKT, 128), jnp.float32),
            pltpu.VMEM((128,), jnp.float32),
            pltpu.VMEM((128,), jnp.int32),
            pltpu.VMEM((256,), jnp.float32),
            pltpu.VMEM((256,), jnp.int32),
            pltpu.VMEM_SHARED((2048,), jnp.float32),
            pltpu.VMEM_SHARED((2048,), jnp.int32),
            pltpu.SemaphoreType.DMA,
            pltpu.SemaphoreType.DMA,
        ],
    )
    sval, sidx = sc_kern(y)

    tval, tidx = pl.pallas_call(
        _tc_body,
        grid=(TC_NBLK,),
        in_specs=[pl.BlockSpec((TC_BW, 128), lambda i: (i + TC_BLK0, 0))],
        out_specs=[
            pl.BlockSpec((1, 1, 128), lambda i: (0, 0, 0)),
            pl.BlockSpec((1, 1, 128), lambda i: (0, 0, 0)),
        ],
        out_shape=[
            jax.ShapeDtypeStruct((1, 1, 128), jnp.float32),
            jax.ShapeDtypeStruct((1, 1, 128), jnp.int32),
        ],
    )(y)

    tv = tval.reshape(R)
    ti = tidx.reshape(R)
    sv = sval.reshape(NC, R)
    si = sidx.reshape(NC, R)

    v, i = sv[0], si[0]
    for vb, ib in ((sv[1], si[1]), (tv, ti)):
        t = (vb > v) | ((vb == v) & (ib < i))
        v = jnp.where(t, vb, v)
        i = jnp.where(t, ib, i)
    return i.astype(jnp.int64)


# speedup vs baseline: 6.5729x; 1.5309x over previous
import jax
import jax.numpy as jnp
from jax import lax
from jax.experimental import pallas as pl

R, C = 128, 100000
TC_BW = 4000
TC_NBLK = C // TC_BW
A = 10
SUB = TC_BW // 8

NEG_INF = float("-inf")
BIG = 1 << 30


def _tc_body(y_ref, oval_ref, oidx_ref):
    i = pl.program_id(0)
    tb = i * TC_BW

    accv = [None] * A
    accj = [None] * A
    for j in range(SUB):
        a = j % A
        v = y_ref[pl.ds(j * 8, 8), :]
        if accv[a] is None:
            accv[a] = v
            accj[a] = jnp.full((8, 128), j, jnp.int32)
        else:
            m = v > accv[a]
            accv[a] = jnp.where(m, v, accv[a])
            accj[a] = jnp.where(m, jnp.int32(j), accj[a])

    bv, bj = accv[0], accj[0]
    for a in range(1, A):
        t = (accv[a] > bv) | ((accv[a] == bv) & (accj[a] < bj))
        bv = jnp.where(t, accv[a], bv)
        bj = jnp.where(t, accj[a], bj)

    sub = lax.broadcasted_iota(jnp.int32, (8, 128), 0)
    bc = bj * 8 + sub + tb
    vmax = jnp.max(bv, axis=0)
    cand = jnp.where(bv == vmax[None, :], bc, jnp.int32(BIG))
    cmin = jnp.min(cand, axis=0)

    @pl.when(i == 0)
    def _():
        oval_ref[0, 0, :] = vmax
        oidx_ref[0, 0, :] = cmin

    @pl.when(i > 0)
    def _():
        pv = oval_ref[0, 0, :]
        pi = oidx_ref[0, 0, :]
        t = (vmax > pv) | ((vmax == pv) & (cmin < pi))
        oval_ref[0, 0, :] = jnp.where(t, vmax, pv)
        oidx_ref[0, 0, :] = jnp.where(t, cmin, pi)


def kernel(x):
    y = jnp.transpose(x)

    tval, tidx = pl.pallas_call(
        _tc_body,
        grid=(TC_NBLK,),
        in_specs=[pl.BlockSpec((TC_BW, 128), lambda i: (i, 0))],
        out_specs=[
            pl.BlockSpec((1, 1, 128), lambda i: (0, 0, 0)),
            pl.BlockSpec((1, 1, 128), lambda i: (0, 0, 0)),
        ],
        out_shape=[
            jax.ShapeDtypeStruct((1, 1, 128), jnp.float32),
            jax.ShapeDtypeStruct((1, 1, 128), jnp.int32),
        ],
    )(y)
    return tidx.reshape(R).astype(jnp.int64)
